# baseline (speedup 1.0000x reference)
import functools

import jax
import jax.numpy as jnp
from jax import lax
from jax.experimental import pallas as pl
from jax.experimental.pallas import tpu as pltpu
from jax.experimental.pallas import tpu_sc as plsc

VOCAB, ED, HD, OD = 1000000, 32, 64, 32
B, L = 16384, 200

NC, NS = 2, 16
NW = NC * NS
SPW = B // NW
C = 8
NCHUNK = SPW // C
SEG = ((0, 120), (120, 80))
CHUNK_IDX = C * L
INV_L = 1.0 / L


TK = 512
NBLK = VOCAB // TK
REM = VOCAB - NBLK * TK
BPW = -(-NBLK // NW)
NT = (ED // 8) * (TK // 128)


def _sc_transpose(tT, rem_flat):
    mesh = plsc.VectorSubcoreMesh(core_axis_name="c", subcore_axis_name="s")

    @functools.partial(
        pl.kernel,
        out_type=jax.ShapeDtypeStruct((VOCAB * ED,), jnp.float32),
        mesh=mesh,
        compiler_params=pltpu.CompilerParams(
            use_tc_tiling_on_sc=True, needs_layout_passes=False
        ),
        scratch_types=dict(
            tin=[pltpu.VMEM((ED, TK), jnp.float32) for _ in range(2)],
            tout=[pltpu.VMEM((TK * ED,), jnp.float32) for _ in range(2)],
            vrem=pltpu.VMEM((REM * ED,), jnp.float32),
            isem=[pltpu.SemaphoreType.DMA for _ in range(2)],
            osem=[pltpu.SemaphoreType.DMA for _ in range(2)],
            rsem=pltpu.SemaphoreType.DMA,
        ),
    )
    def body(tT_hbm, rem_hbm, out_hbm, tin, tout, vrem, isem, osem, rsem):
        wid = lax.axis_index("s") * NC + lax.axis_index("c")
        lo = lax.iota(jnp.int32, 16)
        sub = lo & 7
        half = lo >> 3
        vld_c = half
        vst_c = sub + half * ED

        def blk(i):
            return (wid + i * NW) * TK

        def tile_rc(t):
            return (t % 4) * 8, (t // 4) * 128

        def start_in(i, p):
            @pl.when(blk(i) < NBLK * TK)
            def _():
                pltpu.async_copy(
                    tT_hbm.at[:, pl.ds(blk(i), TK)], tin[p], isem[p]
                )

        def wait_in(p):
            pltpu.make_async_copy(
                tT_hbm.at[:, pl.ds(0, TK)], tin[p], isem[p]
            ).wait()

        def wait_out(p):
            pltpu.make_async_copy(
                out_hbm.at[pl.ds(0, TK * ED)], tout[p], osem[p]
            ).wait()

        def transpose_block(i, p):
            t_in = tin[p]
            t_out = tout[p]
            hi = lo + 16

            def trow(j, _):
                col = lo * 0 + j
                base = pl.multiple_of(j * ED, ED)
                t_out[pl.ds(base, 16)] = plsc.load_gather(t_in, [lo, col])
                t_out[pl.ds(base + 16, 16)] = plsc.load_gather(t_in, [hi, col])
                return 0

            lax.fori_loop(0, TK, trow, 0, unroll=8)

            pltpu.async_copy(
                tout[p], out_hbm.at[pl.ds(blk(i) * ED, TK * ED)], osem[p]
            )

        start_in(0, 0)
        start_in(1, 1)

        def step(i, p):
            @pl.when(blk(i) < NBLK * TK)
            def _():
                wait_in(p)

                @pl.when(i >= 2)
                def _():
                    wait_out(p)

                transpose_block(i, p)

                start_in(i + 2, p)

        def outer(k, _):
            step(k * 2, 0)
            step(k * 2 + 1, 1)
            return 0

        lax.fori_loop(0, (BPW + 1) // 2, outer, 0)

        wait_out(0)
        wait_out(1)

        @pl.when(wid == NW - 1)
        def _():
            pltpu.async_copy(rem_hbm, vrem, rsem)
            pltpu.make_async_copy(rem_hbm, vrem, rsem).wait()
            pltpu.sync_copy(vrem, out_hbm.at[pl.ds(NBLK * TK * ED, REM * ED)])

    return body(tT, rem_flat)


def _sc_pool(ids, table):
    mesh = plsc.VectorSubcoreMesh(core_axis_name="c", subcore_axis_name="s")

    @functools.partial(
        pl.kernel,
        out_type=jax.ShapeDtypeStruct((B, ED), jnp.float32),
        mesh=mesh,
        compiler_params=pltpu.CompilerParams(use_tc_tiling_on_sc=False),
        scratch_types=dict(
            idx=[pltpu.VMEM((C, L), jnp.int32) for _ in range(2)],
            rows=[pltpu.VMEM((CHUNK_IDX, ED), jnp.float32) for _ in range(2)],
            pooled=pltpu.VMEM((SPW, ED), jnp.float32),
            gsem=[pltpu.SemaphoreType.DMA for _ in range(2)],
            isem=[pltpu.SemaphoreType.DMA for _ in range(2)],
        ),
    )
    def body(ids_hbm, table_hbm, out_hbm, idx, rows, pooled, gsem, isem):
        wid = lax.axis_index("s") * NC + lax.axis_index("c")
        sample0 = wid * SPW

        def start_idx(ci, p):
            base = sample0 + ci * C
            pltpu.async_copy(ids_hbm.at[pl.ds(base, C)], idx[p], isem[p])

        def wait_idx(ci, p):
            pltpu.make_async_copy(
                ids_hbm.at[pl.ds(0, C)], idx[p], isem[p]
            ).wait()

        def fire_gathers(p):
            for s in range(C):
                for off, n in SEG:
                    o = s * L + off
                    pltpu.async_copy(
                        table_hbm.at[idx[p].at[s, pl.ds(off, n)]],
                        rows[p].at[pl.ds(o, n)],
                        gsem[p],
                    )

        def drain_gathers(p):
            pltpu.make_async_copy(
                out_hbm.at[pl.ds(0, CHUNK_IDX)], rows[p], gsem[p]
            ).wait()

        def reduce_chunk(ci, p):
            r = rows[p]
            for s in range(C):
                base = s * L

                def rbody(i, carry):
                    a0, a1, b0, b1, c0, c1, d0, d1 = carry
                    q = base + i * 4
                    a0 += r[q, pl.ds(0, 16)]
                    a1 += r[q, pl.ds(16, 16)]
                    b0 += r[q + 1, pl.ds(0, 16)]
                    b1 += r[q + 1, pl.ds(16, 16)]
                    c0 += r[q + 2, pl.ds(0, 16)]
                    c1 += r[q + 2, pl.ds(16, 16)]
                    d0 += r[q + 3, pl.ds(0, 16)]
                    d1 += r[q + 3, pl.ds(16, 16)]
                    return a0, a1, b0, b1, c0, c1, d0, d1

                z = jnp.zeros((16,), jnp.float32)
                a0, a1, b0, b1, c0, c1, d0, d1 = lax.fori_loop(
                    0, L // 4, rbody, (z, z, z, z, z, z, z, z), unroll=5
                )
                out_row = ci * C + s
                pooled[out_row, pl.ds(0, 16)] = ((a0 + b0) + (c0 + d0)) * INV_L
                pooled[out_row, pl.ds(16, 16)] = ((a1 + b1) + (c1 + d1)) * INV_L

        def step(ci, cur, nxt):
            @pl.when(ci + 1 < NCHUNK)
            def _():
                wait_idx(ci + 1, nxt)
                fire_gathers(nxt)

            drain_gathers(cur)

            @pl.when(ci + 2 < NCHUNK)
            def _():
                start_idx(ci + 2, cur)

            reduce_chunk(ci, cur)

        start_idx(0, 0)
        start_idx(1, 1)
        wait_idx(0, 0)
        fire_gathers(0)

        def outer(k, _):
            ci = k * 2
            step(ci, 0, 1)
            step(ci + 1, 1, 0)
            return 0

        lax.fori_loop(0, NCHUNK // 2, outer, 0)

        pltpu.sync_copy(pooled, out_hbm.at[pl.ds(sample0, SPW)])

    return body(ids, table)


def _tc_mlp(x, W1, b1, W2, b2):
    def body(x_ref, w1_ref, b1_ref, w2_ref, b2_ref, o_ref):
        h = jnp.dot(x_ref[...], w1_ref[...], preferred_element_type=jnp.float32)
        h = jnp.maximum(h + b1_ref[...], 0.0)
        o = jnp.dot(h, w2_ref[...], preferred_element_type=jnp.float32)
        o_ref[...] = o + b2_ref[...]

    return pl.pallas_call(
        body,
        out_shape=jax.ShapeDtypeStruct((B, OD), jnp.float32),
    )(x, W1, b1.reshape(1, HD), W2, b2.reshape(1, OD))


def kernel(input_ids, emb_table, W1, b1, W2, b2):
    table_lin = _sc_transpose(emb_table.T, emb_table[VOCAB - REM:].reshape(-1))
    pooled = _sc_pool(input_ids, table_lin.reshape(VOCAB, ED))
    return _tc_mlp(pooled, W1, b1, W2, b2)


# speedup vs baseline: 1.0034x; 1.0034x over previous
import functools

import jax
import jax.numpy as jnp
from jax import lax
from jax.experimental import pallas as pl
from jax.experimental.pallas import tpu as pltpu
from jax.experimental.pallas import tpu_sc as plsc

VOCAB, ED, HD, OD = 1000000, 32, 64, 32
B, L = 16384, 200

NC, NS = 2, 16
NW = NC * NS
SPW = B // NW
C = 8
NCHUNK = SPW // C
SEG = ((0, 120), (120, 80))
CHUNK_IDX = C * L
INV_L = 1.0 / L


TK = 768
NBLK = VOCAB // TK
REM = VOCAB - NBLK * TK
BPW = -(-NBLK // NW)
NT = (ED // 8) * (TK // 128)


def _sc_transpose(tT, rem_flat):
    mesh = plsc.VectorSubcoreMesh(core_axis_name="c", subcore_axis_name="s")

    @functools.partial(
        pl.kernel,
        out_type=jax.ShapeDtypeStruct((VOCAB * ED,), jnp.float32),
        mesh=mesh,
        compiler_params=pltpu.CompilerParams(
            use_tc_tiling_on_sc=True, needs_layout_passes=False
        ),
        scratch_types=dict(
            tin=[pltpu.VMEM((ED, TK), jnp.float32) for _ in range(2)],
            tout=[pltpu.VMEM((TK * ED,), jnp.float32) for _ in range(2)],
            vrem=pltpu.VMEM((REM * ED,), jnp.float32),
            isem=[pltpu.SemaphoreType.DMA for _ in range(2)],
            osem=[pltpu.SemaphoreType.DMA for _ in range(2)],
            rsem=pltpu.SemaphoreType.DMA,
        ),
    )
    def body(tT_hbm, rem_hbm, out_hbm, tin, tout, vrem, isem, osem, rsem):
        wid = lax.axis_index("s") * NC + lax.axis_index("c")
        lo = lax.iota(jnp.int32, 16)
        sub = lo & 7
        half = lo >> 3
        vld_c = half
        vst_c = sub + half * ED

        def blk(i):
            return (wid + i * NW) * TK

        def tile_rc(t):
            return (t % 4) * 8, (t // 4) * 128

        def start_in(i, p):
            @pl.when(blk(i) < NBLK * TK)
            def _():
                for k in range(ED // 8):
                    pltpu.async_copy(
                        tT_hbm.at[pl.ds(k * 8, 8), pl.ds(blk(i), TK)],
                        tin[p].at[pl.ds(k * 8, 8)],
                        isem[p],
                    )

        def wait_in(p):
            for k in range(ED // 8):
                pltpu.make_async_copy(
                    tT_hbm.at[pl.ds(k * 8, 8), pl.ds(0, TK)],
                    tin[p].at[pl.ds(k * 8, 8)],
                    isem[p],
                ).wait()

        def wait_out(p):
            pltpu.make_async_copy(
                out_hbm.at[pl.ds(0, TK * ED)], tout[p], osem[p]
            ).wait()

        def transpose_block(i, p):
            t_in = tin[p]
            t_out = tout[p]
            hi = lo + 16

            def trow(j, _):
                col = lo * 0 + j
                base = pl.multiple_of(j * ED, ED)
                t_out[pl.ds(base, 16)] = plsc.load_gather(t_in, [lo, col])
                t_out[pl.ds(base + 16, 16)] = plsc.load_gather(t_in, [hi, col])
                return 0

            lax.fori_loop(0, TK, trow, 0, unroll=8)

            pltpu.async_copy(
                tout[p], out_hbm.at[pl.ds(blk(i) * ED, TK * ED)], osem[p]
            )

        start_in(0, 0)
        start_in(1, 1)

        def step(i, p):
            @pl.when(blk(i) < NBLK * TK)
            def _():
                wait_in(p)

                @pl.when(i >= 2)
                def _():
                    wait_out(p)

                transpose_block(i, p)

                start_in(i + 2, p)

        def outer(k, _):
            step(k * 2, 0)
            step(k * 2 + 1, 1)
            return 0

        lax.fori_loop(0, (BPW + 1) // 2, outer, 0)

        wait_out(0)
        wait_out(1)

        @pl.when(wid == NW - 1)
        def _():
            pltpu.async_copy(rem_hbm, vrem, rsem)
            pltpu.make_async_copy(rem_hbm, vrem, rsem).wait()
            pltpu.sync_copy(vrem, out_hbm.at[pl.ds(NBLK * TK * ED, REM * ED)])

    return body(tT, rem_flat)


def _sc_pool(ids, table):
    mesh = plsc.VectorSubcoreMesh(core_axis_name="c", subcore_axis_name="s")

    @functools.partial(
        pl.kernel,
        out_type=jax.ShapeDtypeStruct((B, ED), jnp.float32),
        mesh=mesh,
        compiler_params=pltpu.CompilerParams(use_tc_tiling_on_sc=False),
        scratch_types=dict(
            idx=[pltpu.VMEM((C, L), jnp.int32) for _ in range(2)],
            rows=[pltpu.VMEM((CHUNK_IDX, ED), jnp.float32) for _ in range(2)],
            pooled=pltpu.VMEM((SPW, ED), jnp.float32),
            gsem=[pltpu.SemaphoreType.DMA for _ in range(2)],
            isem=[pltpu.SemaphoreType.DMA for _ in range(2)],
        ),
    )
    def body(ids_hbm, table_hbm, out_hbm, idx, rows, pooled, gsem, isem):
        wid = lax.axis_index("s") * NC + lax.axis_index("c")
        sample0 = wid * SPW

        def start_idx(ci, p):
            base = sample0 + ci * C
            pltpu.async_copy(ids_hbm.at[pl.ds(base, C)], idx[p], isem[p])

        def wait_idx(ci, p):
            pltpu.make_async_copy(
                ids_hbm.at[pl.ds(0, C)], idx[p], isem[p]
            ).wait()

        def fire_gathers(p):
            for s in range(C):
                for off, n in SEG:
                    o = s * L + off
                    pltpu.async_copy(
                        table_hbm.at[idx[p].at[s, pl.ds(off, n)]],
                        rows[p].at[pl.ds(o, n)],
                        gsem[p],
                    )

        def drain_gathers(p):
            pltpu.make_async_copy(
                out_hbm.at[pl.ds(0, CHUNK_IDX)], rows[p], gsem[p]
            ).wait()

        def reduce_chunk(ci, p):
            r = rows[p]
            for s in range(C):
                base = s * L

                def rbody(i, carry):
                    a0, a1, b0, b1, c0, c1, d0, d1 = carry
                    q = base + i * 4
                    a0 += r[q, pl.ds(0, 16)]
                    a1 += r[q, pl.ds(16, 16)]
                    b0 += r[q + 1, pl.ds(0, 16)]
                    b1 += r[q + 1, pl.ds(16, 16)]
                    c0 += r[q + 2, pl.ds(0, 16)]
                    c1 += r[q + 2, pl.ds(16, 16)]
                    d0 += r[q + 3, pl.ds(0, 16)]
                    d1 += r[q + 3, pl.ds(16, 16)]
                    return a0, a1, b0, b1, c0, c1, d0, d1

                z = jnp.zeros((16,), jnp.float32)
                a0, a1, b0, b1, c0, c1, d0, d1 = lax.fori_loop(
                    0, L // 4, rbody, (z, z, z, z, z, z, z, z), unroll=5
                )
                out_row = ci * C + s
                pooled[out_row, pl.ds(0, 16)] = ((a0 + b0) + (c0 + d0)) * INV_L
                pooled[out_row, pl.ds(16, 16)] = ((a1 + b1) + (c1 + d1)) * INV_L

        def step(ci, cur, nxt):
            @pl.when(ci + 1 < NCHUNK)
            def _():
                wait_idx(ci + 1, nxt)
                fire_gathers(nxt)

            drain_gathers(cur)

            @pl.when(ci + 2 < NCHUNK)
            def _():
                start_idx(ci + 2, cur)

            reduce_chunk(ci, cur)

        start_idx(0, 0)
        start_idx(1, 1)
        wait_idx(0, 0)
        fire_gathers(0)

        def outer(k, _):
            ci = k * 2
            step(ci, 0, 1)
            step(ci + 1, 1, 0)
            return 0

        lax.fori_loop(0, NCHUNK // 2, outer, 0)

        pltpu.sync_copy(pooled, out_hbm.at[pl.ds(sample0, SPW)])

    return body(ids, table)


def _tc_mlp(x, W1, b1, W2, b2):
    def body(x_ref, w1_ref, b1_ref, w2_ref, b2_ref, o_ref):
        h = jnp.dot(x_ref[...], w1_ref[...], preferred_element_type=jnp.float32)
        h = jnp.maximum(h + b1_ref[...], 0.0)
        o = jnp.dot(h, w2_ref[...], preferred_element_type=jnp.float32)
        o_ref[...] = o + b2_ref[...]

    return pl.pallas_call(
        body,
        out_shape=jax.ShapeDtypeStruct((B, OD), jnp.float32),
    )(x, W1, b1.reshape(1, HD), W2, b2.reshape(1, OD))


def kernel(input_ids, emb_table, W1, b1, W2, b2):
    table_lin = _sc_transpose(emb_table.T, emb_table[VOCAB - REM:].reshape(-1))
    pooled = _sc_pool(input_ids, table_lin.reshape(VOCAB, ED))
    return _tc_mlp(pooled, W1, b1, W2, b2)


# speedup vs baseline: 1.3905x; 1.3858x over previous
import functools

import jax
import jax.numpy as jnp
from jax import lax
from jax.experimental import pallas as pl
from jax.experimental.pallas import tpu as pltpu
from jax.experimental.pallas import tpu_sc as plsc

VOCAB, ED, HD, OD = 1000000, 32, 64, 32
B, L = 16384, 200

NC, NS = 2, 16
NW = NC * NS
SPW = B // NW
C = 8
NCHUNK = SPW // C
SEG = ((0, 120), (120, 80))
CHUNK_IDX = C * L
INV_L = 1.0 / L


def _sc_pool(ids, table):
    mesh = plsc.VectorSubcoreMesh(core_axis_name="c", subcore_axis_name="s")

    @functools.partial(
        pl.kernel,
        out_type=jax.ShapeDtypeStruct((B, ED), jnp.float32),
        mesh=mesh,
        compiler_params=pltpu.CompilerParams(use_tc_tiling_on_sc=False),
        scratch_types=dict(
            idx=[pltpu.VMEM((C, L), jnp.int32) for _ in range(2)],
            rows=[pltpu.VMEM((CHUNK_IDX, ED), jnp.float32) for _ in range(2)],
            pooled=pltpu.VMEM((SPW, ED), jnp.float32),
            gsem=[pltpu.SemaphoreType.DMA for _ in range(2)],
            isem=[pltpu.SemaphoreType.DMA for _ in range(2)],
        ),
    )
    def body(ids_hbm, table_hbm, out_hbm, idx, rows, pooled, gsem, isem):
        wid = lax.axis_index("s") * NC + lax.axis_index("c")
        sample0 = wid * SPW

        def start_idx(ci, p):
            base = sample0 + ci * C
            pltpu.async_copy(ids_hbm.at[pl.ds(base, C)], idx[p], isem[p])

        def wait_idx(ci, p):
            pltpu.make_async_copy(
                ids_hbm.at[pl.ds(0, C)], idx[p], isem[p]
            ).wait()

        def fire_gathers(p):
            for s in range(C):
                for off, n in SEG:
                    o = s * L + off
                    pltpu.async_copy(
                        table_hbm.at[idx[p].at[s, pl.ds(off, n)]],
                        rows[p].at[pl.ds(o, n)],
                        gsem[p],
                    )

        def drain_gathers(p):
            pltpu.make_async_copy(
                out_hbm.at[pl.ds(0, CHUNK_IDX)], rows[p], gsem[p]
            ).wait()

        def reduce_chunk(ci, p):
            r = rows[p]
            for s in range(C):
                base = s * L

                def rbody(i, carry):
                    a0, a1, b0, b1, c0, c1, d0, d1 = carry
                    q = base + i * 4
                    a0 += r[q, pl.ds(0, 16)]
                    a1 += r[q, pl.ds(16, 16)]
                    b0 += r[q + 1, pl.ds(0, 16)]
                    b1 += r[q + 1, pl.ds(16, 16)]
                    c0 += r[q + 2, pl.ds(0, 16)]
                    c1 += r[q + 2, pl.ds(16, 16)]
                    d0 += r[q + 3, pl.ds(0, 16)]
                    d1 += r[q + 3, pl.ds(16, 16)]
                    return a0, a1, b0, b1, c0, c1, d0, d1

                z = jnp.zeros((16,), jnp.float32)
                a0, a1, b0, b1, c0, c1, d0, d1 = lax.fori_loop(
                    0, L // 4, rbody, (z, z, z, z, z, z, z, z), unroll=5
                )
                out_row = ci * C + s
                pooled[out_row, pl.ds(0, 16)] = ((a0 + b0) + (c0 + d0)) * INV_L
                pooled[out_row, pl.ds(16, 16)] = ((a1 + b1) + (c1 + d1)) * INV_L

        def step(ci, cur, nxt):
            @pl.when(ci + 1 < NCHUNK)
            def _():
                wait_idx(ci + 1, nxt)
                fire_gathers(nxt)

            drain_gathers(cur)

            @pl.when(ci + 2 < NCHUNK)
            def _():
                start_idx(ci + 2, cur)

            reduce_chunk(ci, cur)

        start_idx(0, 0)
        start_idx(1, 1)
        wait_idx(0, 0)
        fire_gathers(0)

        def outer(k, _):
            ci = k * 2
            step(ci, 0, 1)
            step(ci + 1, 1, 0)
            return 0

        lax.fori_loop(0, NCHUNK // 2, outer, 0)

        pltpu.sync_copy(pooled, out_hbm.at[pl.ds(sample0, SPW)])

    return body(ids, table)


def _tc_mlp(x, W1, b1, W2, b2):
    def body(x_ref, w1_ref, b1_ref, w2_ref, b2_ref, o_ref):
        h = jnp.dot(x_ref[...], w1_ref[...], preferred_element_type=jnp.float32)
        h = jnp.maximum(h + b1_ref[...], 0.0)
        o = jnp.dot(h, w2_ref[...], preferred_element_type=jnp.float32)
        o_ref[...] = o + b2_ref[...]

    return pl.pallas_call(
        body,
        out_shape=jax.ShapeDtypeStruct((B, OD), jnp.float32),
    )(x, W1, b1.reshape(1, HD), W2, b2.reshape(1, OD))


def kernel(input_ids, emb_table, W1, b1, W2, b2):
    pooled = _sc_pool(input_ids, emb_table)
    return _tc_mlp(pooled, W1, b1, W2, b2)
